# baseline (speedup 1.0000x reference)
import functools

import jax
import jax.numpy as jnp
from jax import lax
from jax.experimental import pallas as pl
from jax.experimental.pallas import tpu as pltpu
from jax.experimental.pallas import tpu_sc as plsc

L = 2048
NCORES = 2
NSUB = 16
NW = NCORES * NSUB
ROWS_PER_W = L // NW
BLK = 16
NBLK = ROWS_PER_W // BLK
LANES = 16
NCH = L // LANES

PARTNER_LUT = 0x4A52


def _body(con_hbm, seq_hbm, out_hbm, seq_v, codes_v, rowp_v, in_v, out_v):
    wid = lax.axis_index("s") * NCORES + lax.axis_index("c")
    row0 = wid * ROWS_PER_W

    pltpu.sync_copy(seq_hbm, seq_v)

    def class_chunk(i, _):
        sl = pl.ds(i * LANES, LANES)
        v = seq_v[0, sl]
        c = jnp.zeros((LANES,), jnp.int32)
        for k in (1, 2, 3):
            s = seq_v[k, sl]
            upd = s > v
            c = jnp.where(upd, k, c)
            v = jnp.maximum(v, s)
        codes_v[sl] = jnp.int32(1) << c
        rowp_v[sl] = (jnp.int32(PARTNER_LUT) >> (c * 4)) & 0xF
        return 0

    lax.fori_loop(0, NCH, class_chunk, 0)

    def block(blk, _):
        rbase = row0 + blk * BLK
        pltpu.sync_copy(con_hbm.at[pl.ds(rbase, BLK)], in_v)
        rowp16 = rowp_v[pl.ds(rbase, LANES)]
        for rr in range(BLK):
            pvec = rowp16.at[jnp.full((LANES,), rr, jnp.int32)].get(
                mode="promise_in_bounds")

            def cols(ci, _):
                sl = pl.ds(ci * LANES, LANES)
                m = (codes_v[sl] & pvec) != 0
                out_v[rr, sl] = jnp.where(m, in_v[rr, sl], 0.0)
                return 0

            lax.fori_loop(0, NCH, cols, 0)
        pltpu.sync_copy(out_v, out_hbm.at[pl.ds(rbase, BLK)])
        return 0

    lax.fori_loop(0, NBLK, block, 0)


@jax.jit
def _canonicalize(con2d, seq):
    mesh = plsc.VectorSubcoreMesh(core_axis_name="c", subcore_axis_name="s")
    f = functools.partial(
        pl.kernel,
        mesh=mesh,
        out_type=jax.ShapeDtypeStruct((L, L), jnp.float32),
        scratch_types=[
            pltpu.VMEM((4, L), jnp.float32),
            pltpu.VMEM((L,), jnp.int32),
            pltpu.VMEM((L,), jnp.int32),
            pltpu.VMEM((BLK, L), jnp.float32),
            pltpu.VMEM((BLK, L), jnp.float32),
        ],
    )(_body)
    return f(con2d, seq)


def kernel(con, feat):
    con2d = con.reshape(L, L)
    seq = feat[0, :4, :, 0]
    out = _canonicalize(con2d, seq)
    return out.reshape(con.shape)


# speedup vs baseline: 1.7123x; 1.7123x over previous
import functools

import jax
import jax.numpy as jnp
from jax import lax
from jax.experimental import pallas as pl
from jax.experimental.pallas import tpu as pltpu
from jax.experimental.pallas import tpu_sc as plsc

L = 2048
NCORES = 2
NSUB = 16
NW = NCORES * NSUB
ROWS_PER_W = L // NW
BLK = 8
NBLK = ROWS_PER_W // BLK
LANES = 16
NCH = L // LANES
UNROLL = 2

PARTNER_LUT = 0x4A52


def _body(con_hbm, seq_hbm, out_hbm, seq_v, codes_v, rowp_v, in_v, out_v,
          sem_in, sem_out):
    wid = lax.axis_index("s") * NCORES + lax.axis_index("c")
    row0 = wid * ROWS_PER_W

    pltpu.sync_copy(seq_hbm, seq_v)

    def class_chunk(i, _):
        sl = pl.ds(i * LANES, LANES)
        v = seq_v[0, sl]
        c = jnp.zeros((LANES,), jnp.int32)
        for k in (1, 2, 3):
            s = seq_v[k, sl]
            upd = s > v
            c = jnp.where(upd, k, c)
            v = jnp.maximum(v, s)
        codes_v[sl] = jnp.int32(1) << c
        rowp_v[sl] = (jnp.int32(PARTNER_LUT) >> (c * 4)) & 0xF
        return 0

    lax.fori_loop(0, NCH, class_chunk, 0)

    idx16 = lax.iota(jnp.int32, LANES)
    ftab = jnp.where(idx16 == 0, 0.0, 1.0).astype(jnp.float32)

    def in_copy(blk, buf):
        return pltpu.make_async_copy(
            con_hbm.at[pl.ds(row0 + blk * BLK, BLK)], in_v.at[buf], sem_in)

    def out_copy(blk, buf):
        return pltpu.make_async_copy(
            out_v.at[buf], out_hbm.at[pl.ds(row0 + blk * BLK, BLK)], sem_out)

    in_copy(0, 0).start()
    for blk in range(NBLK):
        b = blk % 2
        if blk + 1 < NBLK:
            in_copy(blk + 1, (blk + 1) % 2).start()
        in_copy(blk, b).wait()
        if blk >= 2:
            out_copy(blk - 2, b).wait()

        rowp16 = rowp_v[pl.ds(row0 + blk * BLK, LANES)]
        pvecs = [
            rowp16.at[jnp.full((LANES,), rr, jnp.int32)].get(
                mode="promise_in_bounds")
            for rr in range(BLK)
        ]

        def cols(ci, _):
            for u in range(UNROLL):
                sl = pl.ds((ci * UNROLL + u) * LANES, LANES)
                code = codes_v[sl]
                for rr in range(BLK):
                    x = code & pvecs[rr]
                    fm = ftab.at[x].get(mode="promise_in_bounds")
                    out_v[b, rr, sl] = in_v[b, rr, sl] * fm
            return 0

        lax.fori_loop(0, NCH // UNROLL, cols, 0)
        out_copy(blk, b).start()
    out_copy(NBLK - 2, NBLK % 2).wait()
    out_copy(NBLK - 1, (NBLK - 1) % 2).wait()


@jax.jit
def _canonicalize(con2d, seq):
    mesh = plsc.VectorSubcoreMesh(core_axis_name="c", subcore_axis_name="s")
    f = functools.partial(
        pl.kernel,
        mesh=mesh,
        out_type=jax.ShapeDtypeStruct((L, L), jnp.float32),
        scratch_types=[
            pltpu.VMEM((4, L), jnp.float32),
            pltpu.VMEM((L,), jnp.int32),
            pltpu.VMEM((L + LANES,), jnp.int32),
            pltpu.VMEM((2, BLK, L), jnp.float32),
            pltpu.VMEM((2, BLK, L), jnp.float32),
            pltpu.SemaphoreType.DMA,
            pltpu.SemaphoreType.DMA,
        ],
    )(_body)
    return f(con2d, seq)


def kernel(con, feat):
    con2d = con.reshape(L, L)
    seq = feat[0, :4, :, 0]
    out = _canonicalize(con2d, seq)
    return out.reshape(con.shape)


# speedup vs baseline: 2.2034x; 1.2869x over previous
import functools

import jax
import jax.numpy as jnp
from jax import lax
from jax.experimental import pallas as pl
from jax.experimental.pallas import tpu as pltpu
from jax.experimental.pallas import tpu_sc as plsc

L = 2048
NCORES = 2
NSUB = 16
NW = NCORES * NSUB
ROWS_PER_W = L // NW
BLK = 8
NBLK = ROWS_PER_W // BLK
LANES = 16
NCH = L // LANES

PARTNER_LUT = 0x4A52


def _body(con_hbm, seq_hbm, out_hbm, seq_v, codes_v, rowp_v, in_v, out_v,
          sem_in, sem_out):
    wid = lax.axis_index("s") * NCORES + lax.axis_index("c")
    row0 = wid * ROWS_PER_W

    pltpu.sync_copy(seq_hbm, seq_v)

    @plsc.parallel_loop(0, NCH, unroll=2)
    def _class_chunk(i):
        sl = pl.ds(i * LANES, LANES)
        v = seq_v[0, sl]
        c = jnp.zeros((LANES,), jnp.int32)
        for k in (1, 2, 3):
            s = seq_v[k, sl]
            upd = s > v
            c = jnp.where(upd, k, c)
            v = jnp.maximum(v, s)
        codes_v[sl] = jnp.int32(1) << c
        rowp_v[sl] = (jnp.int32(PARTNER_LUT) >> (c * 4)) & 0xF

    idx16 = lax.iota(jnp.int32, LANES)
    ftab = jnp.where(idx16 == 0, 0.0, 1.0).astype(jnp.float32)

    def in_copy(blk, buf):
        return pltpu.make_async_copy(
            con_hbm.at[pl.ds(row0 + blk * BLK, BLK)], in_v.at[buf], sem_in)

    def out_copy(blk, buf):
        return pltpu.make_async_copy(
            out_v.at[buf], out_hbm.at[pl.ds(row0 + blk * BLK, BLK)], sem_out)

    in_copy(0, 0).start()

    def block(blk, _):
        b = blk & 1

        @pl.when(blk + 1 < NBLK)
        def _():
            in_copy(blk + 1, 1 - b).start()

        in_copy(blk, b).wait()

        @pl.when(blk >= 2)
        def _():
            out_copy(blk - 2, b).wait()

        rowp16 = rowp_v[pl.ds(row0 + blk * BLK, LANES)]
        pvecs = [
            rowp16.at[jnp.full((LANES,), rr, jnp.int32)].get(
                mode="promise_in_bounds")
            for rr in range(BLK)
        ]

        @plsc.parallel_loop(0, NCH, unroll=2)
        def _cols(ci):
            sl = pl.ds(ci * LANES, LANES)
            code = codes_v[sl]
            for rr in range(BLK):
                x = code & pvecs[rr]
                fm = ftab.at[x].get(mode="promise_in_bounds")
                out_v[b, rr, sl] = in_v[b, rr, sl] * fm

        out_copy(blk, b).start()
        return 0

    lax.fori_loop(0, NBLK, block, 0)
    out_copy(NBLK - 2, 0).wait()
    out_copy(NBLK - 1, 1).wait()


@jax.jit
def _canonicalize(con2d, seq):
    mesh = plsc.VectorSubcoreMesh(core_axis_name="c", subcore_axis_name="s")
    f = functools.partial(
        pl.kernel,
        mesh=mesh,
        out_type=jax.ShapeDtypeStruct((L, L), jnp.float32),
        scratch_types=[
            pltpu.VMEM((4, L), jnp.float32),
            pltpu.VMEM((L,), jnp.int32),
            pltpu.VMEM((L + LANES,), jnp.int32),
            pltpu.VMEM((2, BLK, L), jnp.float32),
            pltpu.VMEM((2, BLK, L), jnp.float32),
            pltpu.SemaphoreType.DMA,
            pltpu.SemaphoreType.DMA,
        ],
    )(_body)
    return f(con2d, seq)


def kernel(con, feat):
    con2d = con.reshape(L, L)
    seq = feat[0, :4, :, 0]
    out = _canonicalize(con2d, seq)
    return out.reshape(con.shape)


# speedup vs baseline: 2.3085x; 1.0477x over previous
import functools

import jax
import jax.numpy as jnp
from jax import lax
from jax.experimental import pallas as pl
from jax.experimental.pallas import tpu as pltpu
from jax.experimental.pallas import tpu_sc as plsc

L = 2048
NCORES = 2
NSUB = 16
NW = NCORES * NSUB
ROWS_PER_W = L // NW
BLK = 8
NBLK = ROWS_PER_W // BLK
NBUF = 3
LANES = 16
NCH = L // LANES

PARTNER_LUT = 0x4A52


def _body(con_hbm, seq_hbm, out_hbm, seq_v, codes_v, rowp_v, in_v, out_v,
          sem_in, sem_out):
    wid = lax.axis_index("s") * NCORES + lax.axis_index("c")
    row0 = wid * ROWS_PER_W

    def in_copy(blk, buf):
        return pltpu.make_async_copy(
            con_hbm.at[pl.ds(row0 + blk * BLK, BLK)], in_v.at[buf], sem_in)

    def out_copy(blk, buf):
        return pltpu.make_async_copy(
            out_v.at[buf], out_hbm.at[pl.ds(row0 + blk * BLK, BLK)], sem_out)

    for k in range(NBUF):
        in_copy(k, k).start()

    pltpu.sync_copy(seq_hbm, seq_v)

    @plsc.parallel_loop(0, NCH, unroll=2)
    def _class_chunk(i):
        sl = pl.ds(i * LANES, LANES)
        v = seq_v[0, sl]
        c = jnp.zeros((LANES,), jnp.int32)
        for k in (1, 2, 3):
            s = seq_v[k, sl]
            upd = s > v
            c = jnp.where(upd, k, c)
            v = jnp.maximum(v, s)
        codes_v[sl] = jnp.int32(1) << c
        rowp_v[sl] = (jnp.int32(PARTNER_LUT) >> (c * 4)) & 0xF

    idx16 = lax.iota(jnp.int32, LANES)
    ftab = jnp.where(idx16 == 0, 0.0, 1.0).astype(jnp.float32)

    def block(blk, _):
        b = lax.rem(blk, NBUF)
        in_copy(blk, b).wait()

        @pl.when(blk >= NBUF)
        def _():
            out_copy(blk - NBUF, b).wait()

        rowp16 = rowp_v[pl.ds(row0 + blk * BLK, LANES)]
        pvecs = [
            rowp16.at[jnp.full((LANES,), rr, jnp.int32)].get(
                mode="promise_in_bounds")
            for rr in range(BLK)
        ]

        @plsc.parallel_loop(0, NCH, unroll=2)
        def _cols(ci):
            sl = pl.ds(ci * LANES, LANES)
            code = codes_v[sl]
            for rr in range(BLK):
                x = code & pvecs[rr]
                fm = ftab.at[x].get(mode="promise_in_bounds")
                out_v[b, rr, sl] = in_v[b, rr, sl] * fm

        out_copy(blk, b).start()

        @pl.when(blk + NBUF < NBLK)
        def _():
            in_copy(blk + NBUF, b).start()

        return 0

    lax.fori_loop(0, NBLK, block, 0)
    for k in range(NBUF):
        blk = NBLK - NBUF + k
        out_copy(blk, lax.rem(jnp.int32(blk), NBUF)).wait()


@jax.jit
def _canonicalize(con2d, seq):
    mesh = plsc.VectorSubcoreMesh(core_axis_name="c", subcore_axis_name="s")
    f = functools.partial(
        pl.kernel,
        mesh=mesh,
        out_type=jax.ShapeDtypeStruct((L, L), jnp.float32),
        scratch_types=[
            pltpu.VMEM((4, L), jnp.float32),
            pltpu.VMEM((L,), jnp.int32),
            pltpu.VMEM((L + LANES,), jnp.int32),
            pltpu.VMEM((NBUF, BLK, L), jnp.float32),
            pltpu.VMEM((NBUF, BLK, L), jnp.float32),
            pltpu.SemaphoreType.DMA,
            pltpu.SemaphoreType.DMA,
        ],
    )(_body)
    return f(con2d, seq)


def kernel(con, feat):
    con2d = con.reshape(L, L)
    seq = feat[0, :4, :, 0]
    out = _canonicalize(con2d, seq)
    return out.reshape(con.shape)
